# baseline (speedup 1.0000x reference)
import functools

import jax
import jax.numpy as jnp
from jax import lax
from jax.experimental import pallas as pl
from jax.experimental.pallas import tpu as pltpu
from jax.experimental.pallas import tpu_sc as plsc

B = 16
C = 768
HW = 24 * 24
CHW = C * HW
TOT = B * CHW
K = int(0.1 * CHW)
M_DROP = CHW - K

NC, NS = 2, 16
NW = NC * NS
PER_TILE = TOT // NW
POS_PER_TILE = HW // 2
POS_CHUNK = 36
CHUNK = POS_CHUNK * C
NCHUNK = POS_PER_TILE // POS_CHUNK
CB = C // 16

HI_BINS = 1 << 15
LO_BINS = 1 << 16


@functools.cache
def _mesh():
    return plsc.VectorSubcoreMesh(
        core_axis_name="c", subcore_axis_name="s", num_cores=NC, num_subcores=NS)


def _tile_id():
    return lax.axis_index("c") * NS + lax.axis_index("s")


def _splat(ref, idx):
    vec = ref[pl.ds((idx // 16) * 16, 16)]
    return jnp.take_along_axis(vec, jnp.full((16,), idx % 16, jnp.int32),
                               axis=0, mode="promise_in_bounds")


def _zero_fill(ref, n):
    zero16 = jnp.zeros((16,), jnp.int32)

    def body(i, _):
        for u in range(8):
            ref[pl.ds(i * 128 + u * 16, 16)] = zero16
        return 0
    lax.fori_loop(0, n // 128, body, 0)


def _wait_chunk(f_hbm, dst, sem):
    pltpu.make_async_copy(f_hbm.at[pl.ds(0, CHUNK)], dst, sem).wait()


def _wsum_body(w_ref, out_ref):
    w = w_ref[...].astype(jnp.bfloat16).astype(jnp.float32)
    out_ref[...] = jnp.sum(w, axis=1, keepdims=True) / 576.0


def _wsum(W):
    out = pl.pallas_call(
        _wsum_body,
        out_shape=jax.ShapeDtypeStruct((C, 1), jnp.float32),
    )(W)
    return out.reshape(C)


def _hist_hi_body(f_hbm, g_hbm, out_hbm, buf, g_v, hist, sem):
    wid = _tile_id()
    base = wid * PER_TILE
    ones16 = jnp.ones((16,), jnp.int32)

    pltpu.async_copy(f_hbm.at[pl.ds(base, CHUNK)], buf.at[0], sem.at[0])
    _zero_fill(hist, HI_BINS)
    pltpu.sync_copy(g_hbm, g_v)

    def outer(gi, _):
        for bsel in range(2):
            ci = gi * 2 + bsel

            @pl.when(ci + 1 < NCHUNK)
            def _():
                pltpu.async_copy(
                    f_hbm.at[pl.ds(base + (ci + 1) * CHUNK, CHUNK)],
                    buf.at[1 - bsel], sem.at[1 - bsel])

            _wait_chunk(f_hbm, buf.at[bsel], sem.at[bsel])

            def cb_body(cb, _):
                gv = g_v[pl.ds(cb * 16, 16)]
                for p in range(POS_CHUNK):
                    f = buf[bsel, pl.ds(p * C + cb * 16, 16)]
                    prod = f * gv
                    pos = prod > 0.0
                    bits = lax.bitcast_convert_type(prod, jnp.int32)
                    bins = lax.shift_right_logical(bits, 16)
                    plsc.addupdate_scatter(hist, [bins], ones16, mask=pos)
                return 0
            lax.fori_loop(0, CB, cb_body, 0)
        return 0
    lax.fori_loop(0, NCHUNK // 2, outer, 0)

    pltpu.sync_copy(hist.at[pl.ds(0, HI_BINS)], out_hbm.at[wid])


@functools.cache
def _hist_hi():
    return pl.kernel(
        _hist_hi_body,
        out_type=jax.ShapeDtypeStruct((NW, HI_BINS), jnp.int32),
        mesh=_mesh(),
        compiler_params=pltpu.CompilerParams(needs_layout_passes=False),
        scratch_types=[
            pltpu.VMEM((2, CHUNK), jnp.float32),
            pltpu.VMEM((C,), jnp.float32),
            pltpu.VMEM((LO_BINS,), jnp.int32),
            pltpu.SemaphoreType.DMA((2,)),
        ],
    )


def _excl_prefix_search(h, m):
    nb = h.shape[1]
    blk = 128
    nblk = nb // blk
    h3 = h.reshape(B, nblk, blk)
    s = jnp.sum(h3, axis=2)
    iu = lax.broadcasted_iota(jnp.int32, (nblk, nblk), 0)
    ju = lax.broadcasted_iota(jnp.int32, (nblk, nblk), 1)
    U = (iu < ju).astype(jnp.float32)
    pblk = jax.lax.dot(s, U, precision=lax.Precision.HIGHEST)
    iu2 = lax.broadcasted_iota(jnp.int32, (blk, blk), 0)
    ju2 = lax.broadcasted_iota(jnp.int32, (blk, blk), 1)
    U2 = (iu2 < ju2).astype(jnp.float32)
    pin = lax.dot_general(h3, U2, (((2,), (0,)), ((), ())),
                          precision=lax.Precision.HIGHEST)
    pe = pblk[:, :, None] + pin
    le = pe <= m[:, :, None]
    bstar = jnp.sum(le.astype(jnp.int32), axis=(1, 2)) - 1
    pe_at = jnp.max(jnp.where(le, pe, -1.0), axis=(1, 2))
    flat_i = (lax.broadcasted_iota(jnp.int32, (B, nblk, blk), 1) * blk
              + lax.broadcasted_iota(jnp.int32, (B, nblk, blk), 2))
    return bstar[:, None], pe_at[:, None], h3, flat_i


def _scan_hi_body(hist_ref, out_ref):
    h = jnp.sum(hist_ref[...], axis=1).astype(jnp.float32)
    tot = jnp.sum(h, axis=1, keepdims=True)
    col = lax.broadcasted_iota(jnp.int32, (B, HI_BINS), 1)
    h = h + jnp.where(col == 0, float(CHW) - tot, 0.0)
    m = jnp.full((B, 1), float(M_DROP), jnp.float32)
    bstar, pe_at, h3, flat_i = _excl_prefix_search(h, m)
    h_at = jnp.sum(jnp.where(flat_i == bstar[:, :, None], h3, 0.0), axis=(1, 2))
    cnt = h_at[:, None]
    m2 = m - pe_at
    ocol = lax.broadcasted_iota(jnp.int32, (B, 128), 1)
    out = jnp.where(ocol == 0, bstar.astype(jnp.int32),
          jnp.where(ocol == 1, m2.astype(jnp.int32),
          jnp.where(ocol == 2, cnt.astype(jnp.int32), 0)))
    out_ref[...] = out


def _scan_hi(hist):
    return pl.pallas_call(
        _scan_hi_body,
        out_shape=jax.ShapeDtypeStruct((B, 128), jnp.int32),
    )(hist)


def _scan_lo_body(hist_ref, t1_ref, out_ref):
    h = jnp.sum(hist_ref[...], axis=1).astype(jnp.float32)
    cnt = t1_ref[:, 2:3].astype(jnp.float32)
    tot = jnp.sum(h, axis=1, keepdims=True)
    col = lax.broadcasted_iota(jnp.int32, (B, LO_BINS), 1)
    h = h + jnp.where(col == 0, cnt - tot, 0.0)
    m2 = t1_ref[:, 1:2].astype(jnp.float32)
    lstar, _, _, _ = _excl_prefix_search(h, m2)
    tbits = t1_ref[:, 0:1]
    thr_bits = lax.shift_left(tbits, 16) | lstar.astype(jnp.int32)
    thr = lax.bitcast_convert_type(thr_bits, jnp.float32)
    out_ref[...] = jnp.broadcast_to(thr, (B, 128))


def _scan_lo(hist, t1):
    return pl.pallas_call(
        _scan_lo_body,
        out_shape=jax.ShapeDtypeStruct((B, 128), jnp.float32),
    )(hist, t1)


def _hist_lo_body(f_hbm, g_hbm, t_hbm, out_hbm, buf, g_v, t_v, hist, sem):
    wid = _tile_id()
    base = wid * PER_TILE
    batch = wid // 2
    ones16 = jnp.ones((16,), jnp.int32)
    lo_mask = jnp.full((16,), 0xFFFF, jnp.int32)

    pltpu.async_copy(f_hbm.at[pl.ds(base, CHUNK)], buf.at[0], sem.at[0])
    _zero_fill(hist, LO_BINS)
    pltpu.sync_copy(g_hbm, g_v)
    pltpu.sync_copy(t_hbm, t_v)
    tsplat = _splat(t_v, batch)

    def outer(gi, _):
        for bsel in range(2):
            ci = gi * 2 + bsel

            @pl.when(ci + 1 < NCHUNK)
            def _():
                pltpu.async_copy(
                    f_hbm.at[pl.ds(base + (ci + 1) * CHUNK, CHUNK)],
                    buf.at[1 - bsel], sem.at[1 - bsel])

            _wait_chunk(f_hbm, buf.at[bsel], sem.at[bsel])

            def cb_body(cb, _):
                gv = g_v[pl.ds(cb * 16, 16)]
                for p in range(POS_CHUNK):
                    f = buf[bsel, pl.ds(p * C + cb * 16, 16)]
                    prod = f * gv
                    pos = prod > 0.0
                    bits = lax.bitcast_convert_type(prod, jnp.int32)
                    hi = lax.shift_right_logical(bits, 16)
                    sel = pos & (hi == tsplat)
                    lo = bits & lo_mask

                    @pl.when(jnp.any(sel))
                    def _():
                        plsc.addupdate_scatter(hist, [lo], ones16, mask=sel)
                return 0
            lax.fori_loop(0, CB, cb_body, 0)
        return 0
    lax.fori_loop(0, NCHUNK // 2, outer, 0)

    pltpu.sync_copy(hist, out_hbm.at[wid])


@functools.cache
def _hist_lo():
    return pl.kernel(
        _hist_lo_body,
        out_type=jax.ShapeDtypeStruct((NW, LO_BINS), jnp.int32),
        mesh=_mesh(),
        compiler_params=pltpu.CompilerParams(needs_layout_passes=False),
        scratch_types=[
            pltpu.VMEM((2, CHUNK), jnp.float32),
            pltpu.VMEM((C,), jnp.float32),
            pltpu.VMEM((B,), jnp.int32),
            pltpu.VMEM((LO_BINS,), jnp.int32),
            pltpu.SemaphoreType.DMA((2,)),
        ],
    )


def _mask_tc_body(f_ref, g_ref, thr_ref, out_ref):
    f = f_ref[...]
    prod = f * g_ref[...]
    thr = thr_ref[0, 0, 0]
    out_ref[...] = jnp.where(prod <= thr, f, 0.0)


def _mask_tc(f2d, g, thr):
    return pl.pallas_call(
        _mask_tc_body,
        grid=(B,),
        in_specs=[
            pl.BlockSpec((HW, C), lambda b: (b, 0)),
            pl.BlockSpec((1, C), lambda b: (0, 0)),
            pl.BlockSpec((1, 1, 1), lambda b: (b, 0, 0)),
        ],
        out_specs=pl.BlockSpec((HW, C), lambda b: (b, 0)),
        out_shape=jax.ShapeDtypeStruct((B * HW, C), jnp.float32),
    )(f2d, g.reshape(1, C), thr.reshape(B, 1, 1))


def kernel(features, W):
    f2d = jnp.transpose(features, (0, 2, 3, 1)).reshape(B * HW, C)
    f_flat = f2d.reshape(TOT)
    g = _wsum(W)
    hist_a = _hist_hi()(f_flat, g)
    t1 = _scan_hi(hist_a.reshape(B, 2, HI_BINS))
    hist_b = _hist_lo()(f_flat, g, t1[:, 0])
    thr = _scan_lo(hist_b.reshape(B, 2, LO_BINS), t1)[:, 0]
    out = _mask_tc(f2d, g, thr)
    return jnp.transpose(out.reshape(B, 24, 24, C), (0, 3, 1, 2))


# speedup vs baseline: 1.4814x; 1.4814x over previous
import functools

import jax
import jax.numpy as jnp
from jax import lax
from jax.experimental import pallas as pl
from jax.experimental.pallas import tpu as pltpu
from jax.experimental.pallas import tpu_sc as plsc

B = 16
C = 768
HW = 24 * 24
CHW = C * HW
TOT = B * CHW
K = int(0.1 * CHW)
M_DROP = CHW - K

NC, NS = 2, 16
NW = NC * NS
PER_TILE = TOT // NW
POS_PER_TILE = HW // 2
POS_CHUNK = 36
CHUNK = POS_CHUNK * C
NCHUNK = POS_PER_TILE // POS_CHUNK
CB = C // 16

HI_BINS = 1 << 15
LO_BINS = 1 << 16


@functools.cache
def _mesh():
    return plsc.VectorSubcoreMesh(
        core_axis_name="c", subcore_axis_name="s", num_cores=NC, num_subcores=NS)


def _tile_id():
    return lax.axis_index("c") * NS + lax.axis_index("s")


def _splat(ref, idx):
    vec = ref[pl.ds((idx // 16) * 16, 16)]
    return jnp.take_along_axis(vec, jnp.full((16,), idx % 16, jnp.int32),
                               axis=0, mode="promise_in_bounds")


def _zero_fill(ref, n):
    zero16 = jnp.zeros((16,), jnp.int32)

    def body(i, _):
        for u in range(8):
            ref[pl.ds(i * 128 + u * 16, 16)] = zero16
        return 0
    lax.fori_loop(0, n // 128, body, 0)


def _wait_chunk(f_hbm, dst, sem):
    pltpu.make_async_copy(f_hbm.at[pl.ds(0, CHUNK)], dst, sem).wait()


def _wsum_body(w_ref, out_ref):
    w = w_ref[...].astype(jnp.bfloat16).astype(jnp.float32)
    out_ref[...] = jnp.sum(w, axis=1, keepdims=True) / 576.0


def _wsum(W):
    out = pl.pallas_call(
        _wsum_body,
        out_shape=jax.ShapeDtypeStruct((C, 1), jnp.float32),
    )(W)
    return out.reshape(C)


def _hist_hi_body(f_hbm, g_hbm, out_hbm, buf, g_v, hist, hist2, sem):
    wid = _tile_id()
    base = wid * PER_TILE
    ones16 = jnp.ones((16,), jnp.int32)

    pltpu.async_copy(f_hbm.at[pl.ds(base, CHUNK)], buf.at[0], sem.at[0])
    _zero_fill(hist, HI_BINS)
    _zero_fill(hist2, HI_BINS)
    pltpu.sync_copy(g_hbm, g_v)

    def outer(gi, _):
        for bsel in range(2):
            ci = gi * 2 + bsel

            @pl.when(ci + 1 < NCHUNK)
            def _():
                pltpu.async_copy(
                    f_hbm.at[pl.ds(base + (ci + 1) * CHUNK, CHUNK)],
                    buf.at[1 - bsel], sem.at[1 - bsel])

            _wait_chunk(f_hbm, buf.at[bsel], sem.at[bsel])

            def cb_body(cb, _):
                gv = g_v[pl.ds(cb * 16, 16)]
                for p in range(POS_CHUNK):
                    f = buf[bsel, pl.ds(p * C + cb * 16, 16)]
                    prod = f * gv
                    pos = prod > 0.0
                    bits = lax.bitcast_convert_type(prod, jnp.int32)
                    bins = lax.shift_right_logical(bits, 16)
                    bins = jnp.where(pos, bins, 0)
                    plsc.addupdate_scatter(hist if p % 2 == 0 else hist2,
                                           [bins], ones16, mask=pos)
                return 0
            lax.fori_loop(0, CB, cb_body, 0)
        return 0
    lax.fori_loop(0, NCHUNK // 2, outer, 0)

    def merge_body(i, _):
        for u in range(8):
            sl = pl.ds(i * 128 + u * 16, 16)
            hist[sl] = hist[sl] + hist2[sl]
        return 0
    lax.fori_loop(0, HI_BINS // 128, merge_body, 0)

    pltpu.sync_copy(hist, out_hbm.at[wid])


@functools.cache
def _hist_hi():
    return pl.kernel(
        _hist_hi_body,
        out_type=jax.ShapeDtypeStruct((NW, HI_BINS), jnp.int32),
        mesh=_mesh(),
        compiler_params=pltpu.CompilerParams(needs_layout_passes=False),
        scratch_types=[
            pltpu.VMEM((2, CHUNK), jnp.float32),
            pltpu.VMEM((C,), jnp.float32),
            pltpu.VMEM((HI_BINS,), jnp.int32),
            pltpu.VMEM((HI_BINS,), jnp.int32),
            pltpu.SemaphoreType.DMA((2,)),
        ],
    )


def _excl_prefix_search(h, m):
    nb = h.shape[1]
    blk = 128
    nblk = nb // blk
    h3 = h.reshape(B, nblk, blk)
    s = jnp.sum(h3, axis=2)
    iu = lax.broadcasted_iota(jnp.int32, (nblk, nblk), 0)
    ju = lax.broadcasted_iota(jnp.int32, (nblk, nblk), 1)
    U = (iu < ju).astype(jnp.float32)
    pblk = jax.lax.dot(s, U, precision=lax.Precision.HIGHEST)
    iu2 = lax.broadcasted_iota(jnp.int32, (blk, blk), 0)
    ju2 = lax.broadcasted_iota(jnp.int32, (blk, blk), 1)
    U2 = (iu2 < ju2).astype(jnp.float32)
    pin = lax.dot_general(h3, U2, (((2,), (0,)), ((), ())),
                          precision=lax.Precision.HIGHEST)
    pe = pblk[:, :, None] + pin
    le = pe <= m[:, :, None]
    bstar = jnp.sum(le.astype(jnp.int32), axis=(1, 2)) - 1
    pe_at = jnp.max(jnp.where(le, pe, -1.0), axis=(1, 2))
    flat_i = (lax.broadcasted_iota(jnp.int32, (B, nblk, blk), 1) * blk
              + lax.broadcasted_iota(jnp.int32, (B, nblk, blk), 2))
    return bstar[:, None], pe_at[:, None], h3, flat_i


def _scan_hi_body(hist_ref, out_ref):
    h = jnp.sum(hist_ref[...], axis=1).astype(jnp.float32)
    tot = jnp.sum(h, axis=1, keepdims=True)
    col = lax.broadcasted_iota(jnp.int32, (B, HI_BINS), 1)
    h = h + jnp.where(col == 0, float(CHW) - tot, 0.0)
    m = jnp.full((B, 1), float(M_DROP), jnp.float32)
    bstar, pe_at, h3, flat_i = _excl_prefix_search(h, m)
    h_at = jnp.sum(jnp.where(flat_i == bstar[:, :, None], h3, 0.0), axis=(1, 2))
    cnt = h_at[:, None]
    m2 = m - pe_at
    ocol = lax.broadcasted_iota(jnp.int32, (B, 128), 1)
    out = jnp.where(ocol == 0, bstar.astype(jnp.int32),
          jnp.where(ocol == 1, m2.astype(jnp.int32),
          jnp.where(ocol == 2, cnt.astype(jnp.int32), 0)))
    out_ref[...] = out


def _scan_hi(hist):
    return pl.pallas_call(
        _scan_hi_body,
        out_shape=jax.ShapeDtypeStruct((B, 128), jnp.int32),
    )(hist)


def _scan_lo_body(hist_ref, t1_ref, out_ref):
    h = jnp.sum(hist_ref[...], axis=1).astype(jnp.float32)
    cnt = t1_ref[:, 2:3].astype(jnp.float32)
    tot = jnp.sum(h, axis=1, keepdims=True)
    col = lax.broadcasted_iota(jnp.int32, (B, LO_BINS), 1)
    h = h + jnp.where(col == 0, cnt - tot, 0.0)
    m2 = t1_ref[:, 1:2].astype(jnp.float32)
    lstar, _, _, _ = _excl_prefix_search(h, m2)
    tbits = t1_ref[:, 0:1]
    thr_bits = lax.shift_left(tbits, 16) | lstar.astype(jnp.int32)
    thr = lax.bitcast_convert_type(thr_bits, jnp.float32)
    out_ref[...] = jnp.broadcast_to(thr, (B, 128))


def _scan_lo(hist, t1):
    return pl.pallas_call(
        _scan_lo_body,
        out_shape=jax.ShapeDtypeStruct((B, 128), jnp.float32),
    )(hist, t1)


def _hist_lo_body(f_hbm, g_hbm, t_hbm, out_hbm, buf, g_v, t_v, hist, sem):
    wid = _tile_id()
    base = wid * PER_TILE
    batch = wid // 2
    ones16 = jnp.ones((16,), jnp.int32)
    lo_mask = jnp.full((16,), 0xFFFF, jnp.int32)

    pltpu.async_copy(f_hbm.at[pl.ds(base, CHUNK)], buf.at[0], sem.at[0])
    _zero_fill(hist, LO_BINS)
    pltpu.sync_copy(g_hbm, g_v)
    pltpu.sync_copy(t_hbm, t_v)
    tsplat = _splat(t_v, batch)

    def outer(gi, _):
        for bsel in range(2):
            ci = gi * 2 + bsel

            @pl.when(ci + 1 < NCHUNK)
            def _():
                pltpu.async_copy(
                    f_hbm.at[pl.ds(base + (ci + 1) * CHUNK, CHUNK)],
                    buf.at[1 - bsel], sem.at[1 - bsel])

            _wait_chunk(f_hbm, buf.at[bsel], sem.at[bsel])

            def cb_body(cb, _):
                gv = g_v[pl.ds(cb * 16, 16)]
                for p in range(POS_CHUNK):
                    f = buf[bsel, pl.ds(p * C + cb * 16, 16)]
                    prod = f * gv
                    pos = prod > 0.0
                    bits = lax.bitcast_convert_type(prod, jnp.int32)
                    hi = lax.shift_right_logical(bits, 16)
                    sel = pos & (hi == tsplat)
                    lo = bits & lo_mask
                    plsc.addupdate_scatter(hist, [lo], ones16, mask=sel)
                return 0
            lax.fori_loop(0, CB, cb_body, 0)
        return 0
    lax.fori_loop(0, NCHUNK // 2, outer, 0)

    pltpu.sync_copy(hist, out_hbm.at[wid])


@functools.cache
def _hist_lo():
    return pl.kernel(
        _hist_lo_body,
        out_type=jax.ShapeDtypeStruct((NW, LO_BINS), jnp.int32),
        mesh=_mesh(),
        compiler_params=pltpu.CompilerParams(needs_layout_passes=False),
        scratch_types=[
            pltpu.VMEM((2, CHUNK), jnp.float32),
            pltpu.VMEM((C,), jnp.float32),
            pltpu.VMEM((B,), jnp.int32),
            pltpu.VMEM((LO_BINS,), jnp.int32),
            pltpu.SemaphoreType.DMA((2,)),
        ],
    )


def _mask_tc_body(f_ref, g_ref, thr_ref, out_ref):
    f = f_ref[...]
    prod = f * g_ref[...]
    thr = thr_ref[0, 0, 0]
    out_ref[...] = jnp.where(prod <= thr, f, 0.0)


def _mask_tc(f2d, g, thr):
    return pl.pallas_call(
        _mask_tc_body,
        grid=(B,),
        in_specs=[
            pl.BlockSpec((HW, C), lambda b: (b, 0)),
            pl.BlockSpec((1, C), lambda b: (0, 0)),
            pl.BlockSpec((1, 1, 1), lambda b: (b, 0, 0)),
        ],
        out_specs=pl.BlockSpec((HW, C), lambda b: (b, 0)),
        out_shape=jax.ShapeDtypeStruct((B * HW, C), jnp.float32),
    )(f2d, g.reshape(1, C), thr.reshape(B, 1, 1))


def kernel(features, W):
    f2d = jnp.transpose(features, (0, 2, 3, 1)).reshape(B * HW, C)
    f_flat = f2d.reshape(TOT)
    g = _wsum(W)
    hist_a = _hist_hi()(f_flat, g)
    t1 = _scan_hi(hist_a.reshape(B, 2, HI_BINS))
    hist_b = _hist_lo()(f_flat, g, t1[:, 0])
    thr = _scan_lo(hist_b.reshape(B, 2, LO_BINS), t1)[:, 0]
    out = _mask_tc(f2d, g, thr)
    return jnp.transpose(out.reshape(B, 24, 24, C), (0, 3, 1, 2))


# speedup vs baseline: 1.7249x; 1.1643x over previous
import functools

import jax
import jax.numpy as jnp
from jax import lax
from jax.experimental import pallas as pl
from jax.experimental.pallas import tpu as pltpu
from jax.experimental.pallas import tpu_sc as plsc

B = 16
C = 768
HW = 24 * 24
CHW = C * HW
TOT = B * CHW
K = int(0.1 * CHW)
M_DROP = CHW - K

NC, NS = 2, 16
NW = NC * NS
PER_TILE = TOT // NW
POS_PER_TILE = HW // 2
POS_CHUNK = 36
CHUNK = POS_CHUNK * C
NCHUNK = POS_PER_TILE // POS_CHUNK
CB = C // 16
GRP = 9

HI_BINS = 1 << 15
LO_BINS = 1 << 16


@functools.cache
def _mesh():
    return plsc.VectorSubcoreMesh(
        core_axis_name="c", subcore_axis_name="s", num_cores=NC, num_subcores=NS)


def _tile_id():
    return lax.axis_index("c") * NS + lax.axis_index("s")


def _splat(ref, idx):
    vec = ref[pl.ds((idx // 16) * 16, 16)]
    return jnp.take_along_axis(vec, jnp.full((16,), idx % 16, jnp.int32),
                               axis=0, mode="promise_in_bounds")


def _zero_fill(ref, n):
    zero16 = jnp.zeros((16,), jnp.int32)

    def body(i, _):
        for u in range(8):
            ref[pl.ds(i * 128 + u * 16, 16)] = zero16
        return 0
    lax.fori_loop(0, n // 128, body, 0)


def _wait_chunk(f_hbm, dst, sem):
    pltpu.make_async_copy(f_hbm.at[pl.ds(0, CHUNK)], dst, sem).wait()


def _wsum_body(w_ref, out_ref):
    w = w_ref[...].astype(jnp.bfloat16).astype(jnp.float32)
    out_ref[...] = jnp.sum(w, axis=1, keepdims=True) / 576.0


def _wsum(W):
    out = pl.pallas_call(
        _wsum_body,
        out_shape=jax.ShapeDtypeStruct((C, 1), jnp.float32),
    )(W)
    return out.reshape(C)


def _hist_hi_body(f_hbm, g_hbm, out_hbm, buf, g_v, hist, hist2, sem):
    wid = _tile_id()
    base = wid * PER_TILE
    ones16 = jnp.ones((16,), jnp.int32)

    pltpu.async_copy(f_hbm.at[pl.ds(base, CHUNK)], buf.at[0], sem.at[0])
    _zero_fill(hist, HI_BINS)
    _zero_fill(hist2, HI_BINS)
    pltpu.sync_copy(g_hbm, g_v)

    def outer(gi, _):
        for bsel in range(2):
            ci = gi * 2 + bsel

            @pl.when(ci + 1 < NCHUNK)
            def _():
                pltpu.async_copy(
                    f_hbm.at[pl.ds(base + (ci + 1) * CHUNK, CHUNK)],
                    buf.at[1 - bsel], sem.at[1 - bsel])

            _wait_chunk(f_hbm, buf.at[bsel], sem.at[bsel])

            def cb_body(cb, _):
                gv = g_v[pl.ds(cb * 16, 16)]
                for p in range(POS_CHUNK):
                    f = buf[bsel, pl.ds(p * C + cb * 16, 16)]
                    prod = f * gv
                    pos = prod > 0.0
                    bits = lax.bitcast_convert_type(prod, jnp.int32)
                    bins = lax.shift_right_logical(bits, 16)
                    bins = jnp.where(pos, bins, 0)
                    plsc.addupdate_scatter(hist if p % 2 == 0 else hist2,
                                           [bins], ones16, mask=pos)
                return 0
            lax.fori_loop(0, CB, cb_body, 0)
        return 0
    lax.fori_loop(0, NCHUNK // 2, outer, 0)

    def merge_body(i, _):
        for u in range(8):
            sl = pl.ds(i * 128 + u * 16, 16)
            hist[sl] = hist[sl] + hist2[sl]
        return 0
    lax.fori_loop(0, HI_BINS // 128, merge_body, 0)

    pltpu.sync_copy(hist, out_hbm.at[wid])


@functools.cache
def _hist_hi():
    return pl.kernel(
        _hist_hi_body,
        out_type=jax.ShapeDtypeStruct((NW, HI_BINS), jnp.int32),
        mesh=_mesh(),
        compiler_params=pltpu.CompilerParams(needs_layout_passes=False),
        scratch_types=[
            pltpu.VMEM((2, CHUNK), jnp.float32),
            pltpu.VMEM((C,), jnp.float32),
            pltpu.VMEM((HI_BINS,), jnp.int32),
            pltpu.VMEM((HI_BINS,), jnp.int32),
            pltpu.SemaphoreType.DMA((2,)),
        ],
    )


def _excl_prefix_search(h, m):
    nb = h.shape[1]
    blk = 128
    nblk = nb // blk
    h3 = h.reshape(B, nblk, blk)
    s = jnp.sum(h3, axis=2)
    iu = lax.broadcasted_iota(jnp.int32, (nblk, nblk), 0)
    ju = lax.broadcasted_iota(jnp.int32, (nblk, nblk), 1)
    U = (iu < ju).astype(jnp.float32)
    pblk = jax.lax.dot(s, U, precision=lax.Precision.HIGHEST)
    iu2 = lax.broadcasted_iota(jnp.int32, (blk, blk), 0)
    ju2 = lax.broadcasted_iota(jnp.int32, (blk, blk), 1)
    U2 = (iu2 < ju2).astype(jnp.float32)
    pin = lax.dot_general(h3, U2, (((2,), (0,)), ((), ())),
                          precision=lax.Precision.HIGHEST)
    pe = pblk[:, :, None] + pin
    le = pe <= m[:, :, None]
    bstar = jnp.sum(le.astype(jnp.int32), axis=(1, 2)) - 1
    pe_at = jnp.max(jnp.where(le, pe, -1.0), axis=(1, 2))
    flat_i = (lax.broadcasted_iota(jnp.int32, (B, nblk, blk), 1) * blk
              + lax.broadcasted_iota(jnp.int32, (B, nblk, blk), 2))
    return bstar[:, None], pe_at[:, None], h3, flat_i


def _scan_hi_body(hist_ref, out_ref):
    h = jnp.sum(hist_ref[...], axis=1).astype(jnp.float32)
    tot = jnp.sum(h, axis=1, keepdims=True)
    col = lax.broadcasted_iota(jnp.int32, (B, HI_BINS), 1)
    h = h + jnp.where(col == 0, float(CHW) - tot, 0.0)
    m = jnp.full((B, 1), float(M_DROP), jnp.float32)
    bstar, pe_at, h3, flat_i = _excl_prefix_search(h, m)
    h_at = jnp.sum(jnp.where(flat_i == bstar[:, :, None], h3, 0.0), axis=(1, 2))
    cnt = h_at[:, None]
    m2 = m - pe_at
    ocol = lax.broadcasted_iota(jnp.int32, (B, 128), 1)
    out = jnp.where(ocol == 0, bstar.astype(jnp.int32),
          jnp.where(ocol == 1, m2.astype(jnp.int32),
          jnp.where(ocol == 2, cnt.astype(jnp.int32), 0)))
    out_ref[...] = out


def _scan_hi(hist):
    return pl.pallas_call(
        _scan_hi_body,
        out_shape=jax.ShapeDtypeStruct((B, 128), jnp.int32),
    )(hist)


def _scan_lo_body(hist_ref, t1_ref, out_ref):
    h = jnp.sum(hist_ref[...], axis=1).astype(jnp.float32)
    cnt = t1_ref[:, 2:3].astype(jnp.float32)
    tot = jnp.sum(h, axis=1, keepdims=True)
    col = lax.broadcasted_iota(jnp.int32, (B, LO_BINS), 1)
    h = h + jnp.where(col == 0, cnt - tot, 0.0)
    m2 = t1_ref[:, 1:2].astype(jnp.float32)
    lstar, _, _, _ = _excl_prefix_search(h, m2)
    tbits = t1_ref[:, 0:1]
    thr_bits = lax.shift_left(tbits, 16) | lstar.astype(jnp.int32)
    thr = lax.bitcast_convert_type(thr_bits, jnp.float32)
    out_ref[...] = jnp.broadcast_to(thr, (B, 128))


def _scan_lo(hist, t1):
    return pl.pallas_call(
        _scan_lo_body,
        out_shape=jax.ShapeDtypeStruct((B, 128), jnp.float32),
    )(hist, t1)


def _hist_lo_body(f_hbm, g_hbm, t_hbm, out_hbm, buf, g_v, t_v, hist, sem):
    wid = _tile_id()
    base = wid * PER_TILE
    batch = wid // 2
    ones16 = jnp.ones((16,), jnp.int32)
    lo_mask = jnp.full((16,), 0xFFFF, jnp.int32)

    pltpu.async_copy(f_hbm.at[pl.ds(base, CHUNK)], buf.at[0], sem.at[0])
    _zero_fill(hist, LO_BINS)
    pltpu.sync_copy(g_hbm, g_v)
    pltpu.sync_copy(t_hbm, t_v)
    tsplat = _splat(t_v, batch)

    def outer(gi, _):
        for bsel in range(2):
            ci = gi * 2 + bsel

            @pl.when(ci + 1 < NCHUNK)
            def _():
                pltpu.async_copy(
                    f_hbm.at[pl.ds(base + (ci + 1) * CHUNK, CHUNK)],
                    buf.at[1 - bsel], sem.at[1 - bsel])

            _wait_chunk(f_hbm, buf.at[bsel], sem.at[bsel])

            def cb_body(cb, _):
                gv = g_v[pl.ds(cb * 16, 16)]

                def sel_of(p):
                    f = buf[bsel, pl.ds(p * C + cb * 16, 16)]
                    prod = f * gv
                    bits = lax.bitcast_convert_type(prod, jnp.int32)
                    sel = (prod > 0.0) & (
                        lax.shift_right_logical(bits, 16) == tsplat)
                    return sel, bits

                for grp in range(POS_CHUNK // GRP):
                    acc = jnp.zeros((16,), jnp.int32)
                    for pp in range(GRP):
                        sel, _ = sel_of(grp * GRP + pp)
                        acc = acc | sel.astype(jnp.int32)

                    @pl.when(jnp.max(acc) > 0)
                    def _():
                        for pp in range(GRP):
                            sel, bits = sel_of(grp * GRP + pp)
                            plsc.addupdate_scatter(
                                hist, [bits & lo_mask], ones16, mask=sel)
                return 0
            lax.fori_loop(0, CB, cb_body, 0)
        return 0
    lax.fori_loop(0, NCHUNK // 2, outer, 0)

    pltpu.sync_copy(hist, out_hbm.at[wid])


@functools.cache
def _hist_lo():
    return pl.kernel(
        _hist_lo_body,
        out_type=jax.ShapeDtypeStruct((NW, LO_BINS), jnp.int32),
        mesh=_mesh(),
        compiler_params=pltpu.CompilerParams(needs_layout_passes=False),
        scratch_types=[
            pltpu.VMEM((2, CHUNK), jnp.float32),
            pltpu.VMEM((C,), jnp.float32),
            pltpu.VMEM((B,), jnp.int32),
            pltpu.VMEM((LO_BINS,), jnp.int32),
            pltpu.SemaphoreType.DMA((2,)),
        ],
    )


def _mask_tc_body(f_ref, g_ref, thr_ref, out_ref):
    f = f_ref[...]
    prod = f * g_ref[...]
    thr = thr_ref[0, 0, 0]
    out_ref[...] = jnp.where(prod <= thr, f, 0.0)


def _mask_tc(f2d, g, thr):
    return pl.pallas_call(
        _mask_tc_body,
        grid=(B,),
        in_specs=[
            pl.BlockSpec((HW, C), lambda b: (b, 0)),
            pl.BlockSpec((1, C), lambda b: (0, 0)),
            pl.BlockSpec((1, 1, 1), lambda b: (b, 0, 0)),
        ],
        out_specs=pl.BlockSpec((HW, C), lambda b: (b, 0)),
        out_shape=jax.ShapeDtypeStruct((B * HW, C), jnp.float32),
    )(f2d, g.reshape(1, C), thr.reshape(B, 1, 1))


def kernel(features, W):
    f2d = jnp.transpose(features, (0, 2, 3, 1)).reshape(B * HW, C)
    f_flat = f2d.reshape(TOT)
    g = _wsum(W)
    hist_a = _hist_hi()(f_flat, g)
    t1 = _scan_hi(hist_a.reshape(B, 2, HI_BINS))
    hist_b = _hist_lo()(f_flat, g, t1[:, 0])
    thr = _scan_lo(hist_b.reshape(B, 2, LO_BINS), t1)[:, 0]
    out = _mask_tc(f2d, g, thr)
    return jnp.transpose(out.reshape(B, 24, 24, C), (0, 3, 1, 2))


# speedup vs baseline: 1.7996x; 1.0433x over previous
import functools

import jax
import jax.numpy as jnp
from jax import lax
from jax.experimental import pallas as pl
from jax.experimental.pallas import tpu as pltpu
from jax.experimental.pallas import tpu_sc as plsc

B = 16
C = 768
HW = 24 * 24
CHW = C * HW
TOT = B * CHW
K = int(0.1 * CHW)
M_DROP = CHW - K

NC, NS = 2, 16
NW = NC * NS
PER_TILE = TOT // NW
POS_PER_TILE = HW // 2
POS_CHUNK = 36
CHUNK = POS_CHUNK * C
NCHUNK = POS_PER_TILE // POS_CHUNK
CB = C // 16
GRP = 9

HI_BINS = 1 << 15
LO_BINS = 1 << 16


@functools.cache
def _mesh():
    return plsc.VectorSubcoreMesh(
        core_axis_name="c", subcore_axis_name="s", num_cores=NC, num_subcores=NS)


def _tile_id():
    return lax.axis_index("c") * NS + lax.axis_index("s")


def _splat(ref, idx):
    vec = ref[pl.ds((idx // 16) * 16, 16)]
    return jnp.take_along_axis(vec, jnp.full((16,), idx % 16, jnp.int32),
                               axis=0, mode="promise_in_bounds")


def _zero_fill(ref, n):
    zero16 = jnp.zeros((16,), jnp.int32)

    def body(i, _):
        for u in range(8):
            ref[pl.ds(i * 128 + u * 16, 16)] = zero16
        return 0
    lax.fori_loop(0, n // 128, body, 0)


def _wait_chunk(f_hbm, dst, sem):
    pltpu.make_async_copy(f_hbm.at[pl.ds(0, CHUNK)], dst, sem).wait()


def _wsum_body(w_ref, out_ref):
    w = w_ref[...].astype(jnp.bfloat16).astype(jnp.float32)
    out_ref[...] = jnp.sum(w, axis=1, keepdims=True) / 576.0


def _wsum(W):
    out = pl.pallas_call(
        _wsum_body,
        out_shape=jax.ShapeDtypeStruct((C, 1), jnp.float32),
    )(W)
    return out.reshape(C)


def _hist_hi_body(f_hbm, g_hbm, out_hbm, buf, g_v, hist, hist2, sem):
    wid = _tile_id()
    base = wid * PER_TILE
    ones16 = jnp.ones((16,), jnp.int32)

    pltpu.async_copy(f_hbm.at[pl.ds(base, CHUNK)], buf.at[0], sem.at[0])
    _zero_fill(hist, HI_BINS)
    _zero_fill(hist2, HI_BINS)
    pltpu.sync_copy(g_hbm, g_v)

    def outer(gi, _):
        for bsel in range(2):
            ci = gi * 2 + bsel

            @pl.when(ci + 1 < NCHUNK)
            def _():
                pltpu.async_copy(
                    f_hbm.at[pl.ds(base + (ci + 1) * CHUNK, CHUNK)],
                    buf.at[1 - bsel], sem.at[1 - bsel])

            _wait_chunk(f_hbm, buf.at[bsel], sem.at[bsel])

            def cb_body(cb, _):
                gv = g_v[pl.ds(cb * 16, 16)]
                for p in range(POS_CHUNK):
                    f = buf[bsel, pl.ds(p * C + cb * 16, 16)]
                    prod = f * gv
                    pos = prod > 0.0
                    bits = lax.bitcast_convert_type(prod, jnp.int32)
                    bins = lax.shift_right_logical(bits, 16)
                    bins = jnp.where(pos, bins, 0)
                    plsc.addupdate_scatter(hist if p % 2 == 0 else hist2,
                                           [bins], ones16, mask=pos)
                return 0
            lax.fori_loop(0, CB, cb_body, 0)
        return 0
    lax.fori_loop(0, NCHUNK // 2, outer, 0)

    def merge_body(i, _):
        for u in range(8):
            sl = pl.ds(i * 128 + u * 16, 16)
            hist[sl] = hist[sl] + hist2[sl]
        return 0
    lax.fori_loop(0, HI_BINS // 128, merge_body, 0)

    pltpu.sync_copy(hist, out_hbm.at[wid])


@functools.cache
def _hist_hi():
    return pl.kernel(
        _hist_hi_body,
        out_type=jax.ShapeDtypeStruct((NW, HI_BINS), jnp.int32),
        mesh=_mesh(),
        compiler_params=pltpu.CompilerParams(needs_layout_passes=False),
        scratch_types=[
            pltpu.VMEM((2, CHUNK), jnp.float32),
            pltpu.VMEM((C,), jnp.float32),
            pltpu.VMEM((HI_BINS,), jnp.int32),
            pltpu.VMEM((HI_BINS,), jnp.int32),
            pltpu.SemaphoreType.DMA((2,)),
        ],
    )


def _excl_prefix_search(h, m):
    nb_b, nb = h.shape
    blk = 128
    nblk = nb // blk
    h3 = h.reshape(nb_b, nblk, blk)
    s = jnp.sum(h3, axis=2)
    iu = lax.broadcasted_iota(jnp.int32, (nblk, nblk), 0)
    ju = lax.broadcasted_iota(jnp.int32, (nblk, nblk), 1)
    U = (iu < ju).astype(jnp.float32)
    pblk = jax.lax.dot(s, U, precision=lax.Precision.HIGHEST)
    iu2 = lax.broadcasted_iota(jnp.int32, (blk, blk), 0)
    ju2 = lax.broadcasted_iota(jnp.int32, (blk, blk), 1)
    U2 = (iu2 < ju2).astype(jnp.float32)
    pin = lax.dot_general(h3, U2, (((2,), (0,)), ((), ())),
                          precision=lax.Precision.HIGHEST)
    pe = pblk[:, :, None] + pin
    le = pe <= m[:, :, None]
    bstar = jnp.sum(le.astype(jnp.int32), axis=(1, 2)) - 1
    pe_at = jnp.max(jnp.where(le, pe, -1.0), axis=(1, 2))
    flat_i = (lax.broadcasted_iota(jnp.int32, (nb_b, nblk, blk), 1) * blk
              + lax.broadcasted_iota(jnp.int32, (nb_b, nblk, blk), 2))
    return bstar[:, None], pe_at[:, None], h3, flat_i


def _scan_hi_body(hist_ref, out_ref):
    h = jnp.sum(hist_ref[...], axis=1).astype(jnp.float32)
    tot = jnp.sum(h, axis=1, keepdims=True)
    col = lax.broadcasted_iota(jnp.int32, (B, HI_BINS), 1)
    h = h + jnp.where(col == 0, float(CHW) - tot, 0.0)
    m = jnp.full((B, 1), float(M_DROP), jnp.float32)
    bstar, pe_at, h3, flat_i = _excl_prefix_search(h, m)
    h_at = jnp.sum(jnp.where(flat_i == bstar[:, :, None], h3, 0.0), axis=(1, 2))
    cnt = h_at[:, None]
    m2 = m - pe_at
    ocol = lax.broadcasted_iota(jnp.int32, (B, 128), 1)
    out = jnp.where(ocol == 0, bstar.astype(jnp.int32),
          jnp.where(ocol == 1, m2.astype(jnp.int32),
          jnp.where(ocol == 2, cnt.astype(jnp.int32), 0)))
    out_ref[...] = out


def _scan_hi(hist):
    return pl.pallas_call(
        _scan_hi_body,
        out_shape=jax.ShapeDtypeStruct((B, 128), jnp.int32),
    )(hist)


def _hist_lo_body(f_hbm, g_hbm, t_hbm, out_hbm, buf, g_v, t_v, hist, sem):
    wid = _tile_id()
    base = wid * PER_TILE
    batch = wid // 2
    ones16 = jnp.ones((16,), jnp.int32)
    lo_mask = jnp.full((16,), 0xFFFF, jnp.int32)

    pltpu.async_copy(f_hbm.at[pl.ds(base, CHUNK)], buf.at[0], sem.at[0])
    _zero_fill(hist, LO_BINS)
    pltpu.sync_copy(g_hbm, g_v)
    pltpu.sync_copy(t_hbm, t_v)
    tsplat = _splat(t_v, batch)

    def outer(gi, _):
        for bsel in range(2):
            ci = gi * 2 + bsel

            @pl.when(ci + 1 < NCHUNK)
            def _():
                pltpu.async_copy(
                    f_hbm.at[pl.ds(base + (ci + 1) * CHUNK, CHUNK)],
                    buf.at[1 - bsel], sem.at[1 - bsel])

            _wait_chunk(f_hbm, buf.at[bsel], sem.at[bsel])

            def cb_body(cb, _):
                gv = g_v[pl.ds(cb * 16, 16)]

                def sel_of(p):
                    f = buf[bsel, pl.ds(p * C + cb * 16, 16)]
                    prod = f * gv
                    bits = lax.bitcast_convert_type(prod, jnp.int32)
                    sel = (prod > 0.0) & (
                        lax.shift_right_logical(bits, 16) == tsplat)
                    return sel, bits

                for grp in range(POS_CHUNK // GRP):
                    acc = jnp.zeros((16,), jnp.int32)
                    for pp in range(GRP):
                        sel, _ = sel_of(grp * GRP + pp)
                        acc = acc | sel.astype(jnp.int32)

                    @pl.when(jnp.max(acc) > 0)
                    def _():
                        for pp in range(GRP):
                            sel, bits = sel_of(grp * GRP + pp)
                            plsc.addupdate_scatter(
                                hist, [bits & lo_mask], ones16, mask=sel)
                return 0
            lax.fori_loop(0, CB, cb_body, 0)
        return 0
    lax.fori_loop(0, NCHUNK // 2, outer, 0)

    pltpu.sync_copy(hist, out_hbm.at[wid])


@functools.cache
def _hist_lo():
    return pl.kernel(
        _hist_lo_body,
        out_type=jax.ShapeDtypeStruct((NW, LO_BINS), jnp.int32),
        mesh=_mesh(),
        compiler_params=pltpu.CompilerParams(needs_layout_passes=False),
        scratch_types=[
            pltpu.VMEM((2, CHUNK), jnp.float32),
            pltpu.VMEM((C,), jnp.float32),
            pltpu.VMEM((B,), jnp.int32),
            pltpu.VMEM((LO_BINS,), jnp.int32),
            pltpu.SemaphoreType.DMA((2,)),
        ],
    )


def _mask_tc_body(hist_ref, t1_ref, f_ref, g_ref, out_ref):
    h = jnp.sum(hist_ref[0], axis=0, keepdims=True).astype(jnp.float32)
    t1 = t1_ref[0]
    cnt = t1[:, 2:3].astype(jnp.float32)
    tot = jnp.sum(h, axis=1, keepdims=True)
    col = lax.broadcasted_iota(jnp.int32, (1, LO_BINS), 1)
    h = h + jnp.where(col == 0, cnt - tot, 0.0)
    m2 = t1[:, 1:2].astype(jnp.float32)
    lstar, _, _, _ = _excl_prefix_search(h, m2)
    thr_bits = lax.shift_left(t1[:, 0:1], 16) | lstar.astype(jnp.int32)
    thr = lax.bitcast_convert_type(thr_bits, jnp.float32)[0, 0]
    f = f_ref[...]
    out_ref[...] = jnp.where(f * g_ref[...] <= thr, f, 0.0)


def _mask_tc(hist_b, t1, f2d, g):
    return pl.pallas_call(
        _mask_tc_body,
        grid=(B,),
        in_specs=[
            pl.BlockSpec((1, 2, LO_BINS), lambda b: (b, 0, 0)),
            pl.BlockSpec((1, 1, 128), lambda b: (b, 0, 0)),
            pl.BlockSpec((HW, C), lambda b: (b, 0)),
            pl.BlockSpec((1, C), lambda b: (0, 0)),
        ],
        out_specs=pl.BlockSpec((HW, C), lambda b: (b, 0)),
        out_shape=jax.ShapeDtypeStruct((B * HW, C), jnp.float32),
    )(hist_b.reshape(B, 2, LO_BINS), t1.reshape(B, 1, 128),
      f2d, g.reshape(1, C))


def kernel(features, W):
    f2d = jnp.transpose(features, (0, 2, 3, 1)).reshape(B * HW, C)
    f_flat = f2d.reshape(TOT)
    g = _wsum(W)
    hist_a = _hist_hi()(f_flat, g)
    t1 = _scan_hi(hist_a.reshape(B, 2, HI_BINS))
    hist_b = _hist_lo()(f_flat, g, t1[:, 0])
    out = _mask_tc(hist_b, t1, f2d, g)
    return jnp.transpose(out.reshape(B, 24, 24, C), (0, 3, 1, 2))
